# Initial kernel scaffold; baseline (speedup 1.0000x reference)
import jax
import jax.numpy as jnp
from jax.experimental import pallas as pl


def kernel(x, A_indices, A_values, shape, W1, gamma1, beta1, W2, gamma2, beta2, Wout, bout):
    raise NotImplementedError("write your pallas kernel here")


# speedup vs baseline: 4.3475x; 4.3475x over previous
import functools

import jax
import jax.numpy as jnp
from jax import lax
from jax.experimental import pallas as pl
from jax.experimental.pallas import tpu as pltpu
from jax.experimental.pallas import tpu_sc as plsc

EPS = 1e-5
LANES = 16
NC = 2
NS = 16
NW = NC * NS


def _largest_divisor(n, cap):
    for c in range(cap, 0, -1):
        if n % c == 0:
            return c
    return 1


@functools.lru_cache(maxsize=None)
def _make_spmm(n_pad, d, e_pad, ch):
    nch = e_pad // (NW * ch)
    npr = n_pad // NS
    zr = 16
    nzc = n_pad // zr
    nzk = (nzc + NS - 1) // NS
    nj = d // LANES
    ng = ch // LANES
    mesh = plsc.VectorSubcoreMesh(core_axis_name="c", subcore_axis_name="s",
                                  num_cores=NC, num_subcores=NS)

    @functools.partial(
        pl.kernel,
        out_type=jax.ShapeDtypeStruct((NC, n_pad, d), jnp.float32),
        mesh=mesh,
        scratch_types=[
            pltpu.VMEM((2, ch), jnp.int32),
            pltpu.VMEM((2, ch), jnp.int32),
            pltpu.VMEM((2, ch), jnp.int32),
            pltpu.VMEM((ch,), jnp.float32),
            pltpu.VMEM((ch,), jnp.float32),
            pltpu.VMEM((ch,), jnp.float32),
            pltpu.VMEM((ch,), jnp.int32),
            pltpu.VMEM((ch,), jnp.int32),
            pltpu.VMEM((ch,), jnp.int32),
            pltpu.VMEM((ch, d), jnp.float32),
            pltpu.VMEM((ch, d), jnp.float32),
            pltpu.VMEM((ch, d), jnp.float32),
            pltpu.VMEM((zr, d), jnp.float32),
            pltpu.VMEM_SHARED((n_pad, d), jnp.float32),
            pltpu.SemaphoreType.DMA,
            pltpu.SemaphoreType.DMA,
            pltpu.SemaphoreType.DMA,
            pltpu.SemaphoreType.DMA,
            pltpu.SemaphoreType.DMA,
            pltpu.SemaphoreType.DMA,
            pltpu.SemaphoreType.DMA,
            pltpu.SemaphoreType.DMA,
            pltpu.SemaphoreType.DMA,
            pltpu.SemaphoreType.DMA,
        ],
    )
    def spmm(dense_hbm, pk_hbm, val_hbm, out_hbm,
             pb0, pb1, pb2, vb0, vb1, vb2, rc0, rc1, rc2,
             rows0, rows1, rows2, zbuf, acc,
             sem_z, sem_i0, sem_i1, sem_i2,
             sem_g0, sem_g1, sem_g2, sem_s0, sem_s1, sem_s2):
        c = lax.axis_index("c")
        s = lax.axis_index("s")
        wid = c * NS + s

        pb = (pb0, pb1, pb2)
        vb = (vb0, vb1, vb2)
        rc = (rc0, rc1, rc2)
        rows_b = (rows0, rows1, rows2)
        sem_i = (sem_i0, sem_i1, sem_i2)
        sem_g = (sem_g0, sem_g1, sem_g2)
        sem_s = (sem_s0, sem_s1, sem_s2)

        def idx_start(i, t):
            pltpu.async_copy(pk_hbm.at[wid, i], pb[t], sem_i[t])
            pltpu.async_copy(val_hbm.at[wid, i], vb[t], sem_i[t])

        def idx_wait(t):
            pltpu.make_async_copy(pk_hbm.at[0, 0], pb[t], sem_i[t]).wait()
            pltpu.make_async_copy(val_hbm.at[0, 0], vb[t], sem_i[t]).wait()

        def gather_start(t):
            pltpu.async_copy(dense_hbm.at[pb[t].at[1]], rows_b[t], sem_g[t])

        def gather_wait(t):
            pltpu.make_async_copy(dense_hbm.at[pb[0].at[1]],
                                  rows_b[t], sem_g[t]).wait()

        def scatter_start(t):
            pltpu.async_copy(rows_b[t], acc.at[rc[t]], sem_s[t], add=True)

        def scatter_wait(t):
            pltpu.make_async_copy(rows_b[t], acc.at[rc[0]], sem_s[t]).wait()

        def scale(t):
            rb_ = rows_b[t]
            vb_ = vb[t]
            pb_ = pb[t]
            rc_ = rc[t]
            for g in range(ng):
                rc_[pl.ds(g * LANES, LANES)] = pb_[0, pl.ds(g * LANES, LANES)]

            def egroup(g, carry2):
                vals16 = vb_[pl.ds(g * LANES, LANES)]
                for l in range(LANES):
                    v = vals16[l]
                    e_i = g * LANES + l
                    for j in range(nj):
                        sl = rb_[e_i, pl.ds(j * LANES, LANES)]
                        rb_[e_i, pl.ds(j * LANES, LANES)] = sl * v
                return carry2

            lax.fori_loop(0, ng, egroup, 0)

        idx_start(0, 0)
        idx_start(1, 1)

        zv = jnp.zeros((LANES,), jnp.float32)

        def zrow(i, carry):
            for j in range(nj):
                zbuf[i, pl.ds(j * LANES, LANES)] = zv
            return carry

        lax.fori_loop(0, zr, zrow, 0)
        for k in range(nzk):
            cid = s + NS * k

            @pl.when(cid < nzc)
            def _():
                pltpu.async_copy(
                    zbuf, acc.at[pl.ds(pl.multiple_of(cid * zr, zr), zr)],
                    sem_z)
        for k in range(nzk):
            cid = s + NS * k

            @pl.when(cid < nzc)
            def _():
                pltpu.make_async_copy(zbuf, acc.at[pl.ds(0, zr)], sem_z).wait()

        idx_wait(0)
        gather_start(0)
        plsc.subcore_barrier()

        def step(i, t):
            tn = (t + 1) % 3
            tnn = (t + 2) % 3

            @pl.when(i > 1)
            def _():
                scatter_wait(tn)

            @pl.when(i + 1 < nch)
            def _():
                idx_wait(tn)
                gather_start(tn)

            @pl.when(i + 2 < nch)
            def _():
                idx_start(i + 2, tnn)

            gather_wait(t)
            scale(t)
            scatter_start(t)

        def triple(q, carry):
            i0 = 3 * q
            step(i0, 0)
            step(i0 + 1, 1)
            step(i0 + 2, 2)
            return carry

        lax.fori_loop(0, nch // 3, triple, 0)
        scatter_wait((nch - 2) % 3)
        scatter_wait((nch - 1) % 3)
        plsc.subcore_barrier()
        off = pl.multiple_of(s * npr, 8)
        pltpu.sync_copy(acc.at[pl.ds(off, npr)],
                        out_hbm.at[c, pl.ds(off, npr)])

    return spmm


@functools.lru_cache(maxsize=None)
def _make_mm_stats(n, d, rb):
    nb = n // rb

    def body(p_ref, w_ref, g_ref, b_ref, h_ref, ss_ref):
        i = pl.program_id(0)
        p = p_ref[0] + p_ref[1]
        h = jnp.dot(p, w_ref[...], preferred_element_type=jnp.float32)
        h_ref[...] = h
        rowsel = lax.broadcasted_iota(jnp.int32, (8, d), 0)
        s1 = jnp.sum(h, axis=0)
        s2 = jnp.sum(h * h, axis=0)
        upd = (jnp.where(rowsel == 0, s1[None, :], 0.0)
               + jnp.where(rowsel == 1, s2[None, :], 0.0))

        @pl.when(i == 0)
        def _():
            ss_ref[...] = upd

        @pl.when(i > 0)
        def _():
            ss_ref[...] = ss_ref[...] + upd

        @pl.when(i == nb - 1)
        def _():
            ss = ss_ref[...]
            mean = ss[0] / n
            var = ss[1] / n - mean * mean
            scale = g_ref[0] * lax.rsqrt(var + EPS)
            shift = b_ref[0] - mean * scale
            ss_ref[...] = ss + (jnp.where(rowsel == 2, scale[None, :], 0.0)
                                + jnp.where(rowsel == 3, shift[None, :], 0.0))

    return pl.pallas_call(
        body,
        grid=(nb,),
        in_specs=[
            pl.BlockSpec((NC, rb, d), lambda i: (0, i, 0)),
            pl.BlockSpec((d, d), lambda i: (0, 0)),
            pl.BlockSpec((1, d), lambda i: (0, 0)),
            pl.BlockSpec((1, d), lambda i: (0, 0)),
        ],
        out_specs=[
            pl.BlockSpec((rb, d), lambda i: (i, 0)),
            pl.BlockSpec((8, d), lambda i: (0, 0)),
        ],
        out_shape=[
            jax.ShapeDtypeStruct((n, d), jnp.float32),
            jax.ShapeDtypeStruct((8, d), jnp.float32),
        ],
    )


@functools.lru_cache(maxsize=None)
def _make_bn_relu(n, d, rb):
    nb = n // rb

    def body(h_ref, ss_ref, o_ref):
        ss = ss_ref[...]
        scale = ss[2][None, :]
        shift = ss[3][None, :]
        o_ref[...] = jnp.maximum(h_ref[...] * scale + shift, 0.0)

    return pl.pallas_call(
        body,
        grid=(nb,),
        in_specs=[
            pl.BlockSpec((rb, d), lambda i: (i, 0)),
            pl.BlockSpec((8, d), lambda i: (0, 0)),
        ],
        out_specs=pl.BlockSpec((rb, d), lambda i: (i, 0)),
        out_shape=jax.ShapeDtypeStruct((n, d), jnp.float32),
    )


@functools.lru_cache(maxsize=None)
def _make_bn_relu_head(n, d, rb):
    nb = n // rb

    def body(h_ref, ss_ref, w_ref, b_ref, o_ref):
        ss = ss_ref[...]
        scale = ss[2][None, :]
        shift = ss[3][None, :]
        h = jnp.maximum(h_ref[...] * scale + shift, 0.0)
        y = jnp.sum(h * w_ref[...], axis=1) + b_ref[0, 0]
        o_ref[...] = y[:, None]

    return pl.pallas_call(
        body,
        grid=(nb,),
        in_specs=[
            pl.BlockSpec((rb, d), lambda i: (i, 0)),
            pl.BlockSpec((8, d), lambda i: (0, 0)),
            pl.BlockSpec((1, d), lambda i: (0, 0)),
            pl.BlockSpec((1, 1), lambda i: (0, 0)),
        ],
        out_specs=pl.BlockSpec((rb, 1), lambda i: (i, 0)),
        out_shape=jax.ShapeDtypeStruct((n, 1), jnp.float32),
    )


def kernel(x, A_indices, A_values, shape, W1, gamma1, beta1,
           W2, gamma2, beta2, Wout, bout):
    n, d = x.shape
    e = A_values.shape[0]

    row = A_indices[0].astype(jnp.int32)
    col = A_indices[1].astype(jnp.int32)
    val = A_values.astype(jnp.float32)

    best = None
    for c in range(96, 15, -16):
        unit = 3 * NW * c
        ep = ((e + unit - 1) // unit) * unit
        if best is None or ep < best[0]:
            best = (ep, c)
    e_pad, ch = best
    if e_pad != e:
        pad = e_pad - e
        row = jnp.concatenate([row, jnp.zeros((pad,), jnp.int32)])
        col = jnp.concatenate([col, jnp.zeros((pad,), jnp.int32)])
        val = jnp.concatenate([val, jnp.zeros((pad,), jnp.float32)])
    nch = e_pad // (NW * ch)
    pk = jnp.stack([row.reshape(NW, nch, ch), col.reshape(NW, nch, ch)],
                   axis=2)
    vals = val.reshape(NW, nch, ch)

    n_pad = ((n + NS * 8 - 1) // (NS * 8)) * (NS * 8)
    rb = _largest_divisor(n, 1024)
    spmm = _make_spmm(n_pad, d, e_pad, ch)
    mm_stats = _make_mm_stats(n, d, rb)
    bn_relu = _make_bn_relu(n, d, rb)
    bn_relu_head = _make_bn_relu_head(n, d, rb)

    g1 = gamma1.reshape(1, d)
    b1 = beta1.reshape(1, d)
    g2 = gamma2.reshape(1, d)
    b2 = beta2.reshape(1, d)
    wout = Wout.reshape(1, d)
    bo = bout.reshape(1, 1)

    p = spmm(x, pk, vals)
    h1, ss1 = mm_stats(p, W1, g1, b1)
    h1 = bn_relu(h1, ss1)
    q = spmm(h1, pk, vals)
    h2, ss2 = mm_stats(q, W2, g2, b2)
    y = bn_relu_head(h2, ss2, wout, bo)
    return y[:, 0]
